# baseline (speedup 1.0000x reference)
import dataclasses
import functools

import jax
import jax.numpy as jnp
from jax import lax
from jax.experimental import pallas as pl
from jax.experimental.pallas import tpu as pltpu
from jax.experimental.pallas import tpu_sc as plsc

B = 16384
TABLE = 1001
NC = 2
NS = 16
LANES = 16
NW = NC * NS
B_PER_W = B // NW
OUT_CHUNK = 128
N_CHUNKS = B_PER_W // OUT_CHUNK


@functools.lru_cache(maxsize=None)
def _build_sc_lookup():
    cp = pltpu.CompilerParams()
    if "needs_layout_passes" in pltpu.CompilerParams.__dataclass_fields__:
        cp = dataclasses.replace(cp, needs_layout_passes=False)

    @functools.partial(
        pl.kernel,
        out_type=jax.ShapeDtypeStruct((B,), jnp.float32),
        mesh=plsc.VectorSubcoreMesh(
            core_axis_name="c", subcore_axis_name="s",
            num_cores=NC, num_subcores=NS,
        ),
        scratch_types=[
            pltpu.VMEM((B_PER_W,), jnp.int32),
            pltpu.VMEM((TABLE,), jnp.float32),
            pltpu.VMEM((B_PER_W,), jnp.float32),
            pltpu.SemaphoreType.DMA,
            pltpu.SemaphoreType.DMA,
            pltpu.SemaphoreType.DMA,
        ],
        compiler_params=cp,
    )
    def _sc_lookup(t_hbm, table_hbm, out_hbm, idx_v, tab_v, out_v,
                   sem_i, sem_t, sem_o):
        wid = lax.axis_index("s") * NC + lax.axis_index("c")
        base = wid * B_PER_W
        cp_idx = pltpu.async_copy(t_hbm.at[pl.ds(base, B_PER_W)], idx_v, sem_i)
        cp_tab = pltpu.async_copy(table_hbm, tab_v, sem_t)
        cp_idx.wait()
        cp_tab.wait()

        out_copies = []
        for c in range(N_CHUNKS):
            for i in range(c * OUT_CHUNK, (c + 1) * OUT_CHUNK, LANES):
                idx = idx_v[pl.ds(i, LANES)]
                out_v[pl.ds(i, LANES)] = plsc.load_gather(tab_v, [idx])
            out_copies.append(pltpu.async_copy(
                out_v.at[pl.ds(c * OUT_CHUNK, OUT_CHUNK)],
                out_hbm.at[pl.ds(base + c * OUT_CHUNK, OUT_CHUNK)],
                sem_o))
        for copy in out_copies:
            copy.wait()

    return _sc_lookup


def kernel(t, mask_rate):
    return _build_sc_lookup()(t.astype(jnp.int32), mask_rate)


# speedup vs baseline: 1.0142x; 1.0142x over previous
import dataclasses
import functools

import jax
import jax.numpy as jnp
from jax import lax
from jax.experimental import pallas as pl
from jax.experimental.pallas import tpu as pltpu
from jax.experimental.pallas import tpu_sc as plsc

B = 16384
TABLE = 1001
NC = 2
NS = 16
LANES = 16
NW = NC * NS
B_PER_W = B // NW


@functools.lru_cache(maxsize=None)
def _build_sc_lookup():
    cp = pltpu.CompilerParams()
    if "needs_layout_passes" in pltpu.CompilerParams.__dataclass_fields__:
        cp = dataclasses.replace(cp, needs_layout_passes=False)

    @functools.partial(
        pl.kernel,
        out_type=jax.ShapeDtypeStruct((B,), jnp.float32),
        mesh=plsc.VectorSubcoreMesh(
            core_axis_name="c", subcore_axis_name="s",
            num_cores=NC, num_subcores=NS,
        ),
        scratch_types=[
            pltpu.VMEM((B_PER_W,), jnp.int32),
            pltpu.VMEM((TABLE,), jnp.float32),
            pltpu.VMEM((B_PER_W,), jnp.float32),
            pltpu.SemaphoreType.DMA,
            pltpu.SemaphoreType.DMA,
        ],
        compiler_params=cp,
    )
    def _sc_lookup(t_hbm, table_hbm, out_hbm, idx_v, tab_v, out_v, sem_i, sem_t):
        wid = lax.axis_index("s") * NC + lax.axis_index("c")
        base = wid * B_PER_W
        cp_idx = pltpu.async_copy(t_hbm.at[pl.ds(base, B_PER_W)], idx_v, sem_i)
        cp_tab = pltpu.async_copy(table_hbm, tab_v, sem_t)
        cp_idx.wait()
        cp_tab.wait()

        @plsc.parallel_loop(0, B_PER_W, step=LANES, unroll=4)
        def _(i):
            idx = idx_v[pl.ds(i, LANES)]
            out_v[pl.ds(i, LANES)] = plsc.load_gather(tab_v, [idx])

        pltpu.sync_copy(out_v, out_hbm.at[pl.ds(base, B_PER_W)])

    return _sc_lookup


def kernel(t, mask_rate):
    return _build_sc_lookup()(t.astype(jnp.int32), mask_rate)
